# Initial kernel scaffold; baseline (speedup 1.0000x reference)
import jax
import jax.numpy as jnp
from jax.experimental import pallas as pl


def kernel(x, importance, proj_all_W, proj_all_b, proj_fk_W, proj_fk_b, proj_rk_W, proj_rk_b, neuron_emb):
    raise NotImplementedError("write your pallas kernel here")


# speedup vs baseline: 18.3967x; 18.3967x over previous
import jax
import jax.numpy as jnp
from jax.experimental import pallas as pl

D_MODEL = 2048
D_SPACE = 64
GROUPS = (
    (0, 256, 8),
    (256, 256, 8),
    (512, 256, 8),
    (768, 256, 8),
    (1024, 256, 8),
    (1280, 1024, 4),
    (2304, 1024, 4),
)
N_OUT = 3328
N_PROJ = 448
TILE = 256


def _router_kernel(x_ref, w_ref, b_ref, e_ref, o_ref):
    h = jnp.dot(x_ref[...], w_ref[...], preferred_element_type=jnp.float32)
    h = (h + b_ref[...]).astype(jnp.bfloat16)
    logits = jnp.dot(h, e_ref[...], preferred_element_type=jnp.float32)

    for off, width, k in GROUPS:
        l = logits[:, off:off + width]
        m = jnp.max(l, axis=-1, keepdims=True)
        ex = jnp.exp(l - m)
        z = jnp.sum(ex, axis=-1, keepdims=True)
        w = ex / z
        cur = w
        for _ in range(k - 1):
            mv = jnp.max(cur, axis=-1, keepdims=True)
            cur = jnp.where(cur >= mv, -1.0, cur)
        thresh = jnp.max(cur, axis=-1, keepdims=True)
        sw = jnp.where(w >= thresh, w, 0.0)
        o_ref[:, off:off + width] = sw / (jnp.sum(sw, axis=-1, keepdims=True)
                                          + 1e-08)


@jax.jit
def kernel(x, importance, proj_all_W, proj_all_b, proj_fk_W, proj_fk_b,
           proj_rk_W, proj_rk_b, neuron_emb):
    del importance
    b, s, d = x.shape
    n_tok = b * s
    xb = x.reshape(n_tok, d).astype(jnp.bfloat16)

    w_cat = jnp.concatenate([proj_all_W, proj_fk_W, proj_rk_W],
                            axis=0).T.astype(jnp.bfloat16)
    b_cat = jnp.concatenate([proj_all_b, proj_fk_b, proj_rk_b],
                            axis=0).reshape(1, N_PROJ)

    norm = jnp.maximum(jnp.linalg.norm(neuron_emb, axis=-1, keepdims=True),
                       1e-12)
    emb_norm = neuron_emb / norm

    emb_chunks = (
        emb_norm[0:256], emb_norm[256:512], emb_norm[512:768],
        emb_norm[512:768], emb_norm[768:1024], emb_norm[1024:2048],
        emb_norm[2048:3072],
    )
    e_bd = jnp.zeros((N_PROJ, N_OUT), dtype=jnp.float32)
    for g, (off, width, _) in enumerate(GROUPS):
        e_bd = e_bd.at[64 * g:64 * (g + 1), off:off + width].set(
            emb_chunks[g].T)
    e_bd = e_bd.astype(jnp.bfloat16)

    grid = (n_tok // TILE,)
    out = pl.pallas_call(
        _router_kernel,
        grid=grid,
        in_specs=[
            pl.BlockSpec((TILE, d), lambda i: (i, 0)),
            pl.BlockSpec((d, N_PROJ), lambda i: (0, 0)),
            pl.BlockSpec((1, N_PROJ), lambda i: (0, 0)),
            pl.BlockSpec((N_PROJ, N_OUT), lambda i: (0, 0)),
        ],
        out_specs=pl.BlockSpec((TILE, N_OUT), lambda i: (i, 0)),
        out_shape=jax.ShapeDtypeStruct((n_tok, N_OUT), jnp.float32),
    )(xb, w_cat, b_cat, e_bd)
    return out.reshape(b, s, N_OUT)


# speedup vs baseline: 21.9065x; 1.1908x over previous
import jax
import jax.numpy as jnp
from jax.experimental import pallas as pl

D_MODEL = 2048
D_SPACE = 64
GROUPS = (
    (0, 256, 8),
    (256, 256, 8),
    (512, 256, 8),
    (768, 256, 8),
    (1024, 256, 8),
    (1280, 1024, 4),
    (2304, 1024, 4),
)
N_OUT = 3328
N_PROJ = 448
TILE = 256


def _router_kernel(x_ref, w_ref, b_ref, e_ref, o_ref):
    xb = x_ref[...].astype(jnp.bfloat16)
    h = jnp.dot(xb, w_ref[...], preferred_element_type=jnp.float32)
    h = (h + b_ref[...]).astype(jnp.bfloat16)
    logits = jnp.dot(h, e_ref[...], preferred_element_type=jnp.float32)

    for off, width, k in GROUPS:
        l = logits[:, off:off + width]
        m = jnp.max(l, axis=-1, keepdims=True)
        ex = jnp.exp(l - m)
        z = jnp.sum(ex, axis=-1, keepdims=True)
        w = ex / z
        cur = w
        for _ in range(k - 1):
            mv = jnp.max(cur, axis=-1, keepdims=True)
            cur = jnp.where(cur >= mv, -1.0, cur)
        thresh = jnp.max(cur, axis=-1, keepdims=True)
        sw = jnp.where(w >= thresh, w, 0.0)
        o_ref[:, off:off + width] = sw / (jnp.sum(sw, axis=-1, keepdims=True)
                                          + 1e-08)


@jax.jit
def kernel(x, importance, proj_all_W, proj_all_b, proj_fk_W, proj_fk_b,
           proj_rk_W, proj_rk_b, neuron_emb):
    del importance
    b, s, d = x.shape
    n_tok = b * s
    xf = x.reshape(n_tok, d)

    w_cat = jnp.concatenate([proj_all_W, proj_fk_W, proj_rk_W],
                            axis=0).T.astype(jnp.bfloat16)
    b_cat = jnp.concatenate([proj_all_b, proj_fk_b, proj_rk_b],
                            axis=0).reshape(1, N_PROJ)

    norm = jnp.maximum(jnp.linalg.norm(neuron_emb, axis=-1, keepdims=True),
                       1e-12)
    emb_norm = neuron_emb / norm

    emb_chunks = (
        emb_norm[0:256], emb_norm[256:512], emb_norm[512:768],
        emb_norm[512:768], emb_norm[768:1024], emb_norm[1024:2048],
        emb_norm[2048:3072],
    )
    e_bd = jnp.zeros((N_PROJ, N_OUT), dtype=jnp.float32)
    for g, (off, width, _) in enumerate(GROUPS):
        e_bd = e_bd.at[64 * g:64 * (g + 1), off:off + width].set(
            emb_chunks[g].T)
    e_bd = e_bd.astype(jnp.bfloat16)

    grid = (n_tok // TILE,)
    out = pl.pallas_call(
        _router_kernel,
        grid=grid,
        in_specs=[
            pl.BlockSpec((TILE, d), lambda i: (i, 0)),
            pl.BlockSpec((d, N_PROJ), lambda i: (0, 0)),
            pl.BlockSpec((1, N_PROJ), lambda i: (0, 0)),
            pl.BlockSpec((N_PROJ, N_OUT), lambda i: (0, 0)),
        ],
        out_specs=pl.BlockSpec((TILE, N_OUT), lambda i: (i, 0)),
        out_shape=jax.ShapeDtypeStruct((n_tok, N_OUT), jnp.float32),
    )(xf, w_cat, b_cat, e_bd)
    return out.reshape(b, s, N_OUT)


# speedup vs baseline: 24.5068x; 1.1187x over previous
import jax
import jax.numpy as jnp
from jax.experimental import pallas as pl

D_MODEL = 2048
D_SPACE = 64
GROUPS = (
    (0, 256, 8),
    (256, 256, 8),
    (512, 256, 8),
    (768, 256, 8),
    (1024, 256, 8),
    (1280, 1024, 4),
    (2304, 1024, 4),
)
N_OUT = 3328
N_PROJ = 448
TILE = 256


def _router_kernel(x_ref, w_ref, b_ref, e_ref, o_ref):
    xb = x_ref[...].astype(jnp.bfloat16)
    h = jnp.dot(xb, w_ref[...], preferred_element_type=jnp.float32)
    h = (h + b_ref[...]).astype(jnp.bfloat16)
    logits = jnp.dot(h, e_ref[...], preferred_element_type=jnp.float32)

    for off, width, k in GROUPS:
        l = logits[:, off:off + width]
        m = jnp.max(l, axis=-1, keepdims=True)
        cur, mv = l, m
        for _ in range(k - 1):
            cur = jnp.where(cur >= mv, -jnp.inf, cur)
            mv = jnp.max(cur, axis=-1, keepdims=True)
        ex = jnp.exp(l - m)
        mex = jnp.where(l >= mv, ex, 0.0)
        se = jnp.sum(mex, axis=-1, keepdims=True)
        o_ref[:, off:off + width] = mex * (1.0 / se)


@jax.jit
def kernel(x, importance, proj_all_W, proj_all_b, proj_fk_W, proj_fk_b,
           proj_rk_W, proj_rk_b, neuron_emb):
    del importance
    b, s, d = x.shape
    n_tok = b * s
    xf = x.reshape(n_tok, d)

    w_cat = jnp.concatenate([proj_all_W, proj_fk_W, proj_rk_W],
                            axis=0).T.astype(jnp.bfloat16)
    b_cat = jnp.concatenate([proj_all_b, proj_fk_b, proj_rk_b],
                            axis=0).reshape(1, N_PROJ)

    norm = jnp.maximum(jnp.linalg.norm(neuron_emb, axis=-1, keepdims=True),
                       1e-12)
    emb_norm = neuron_emb / norm

    emb_chunks = (
        emb_norm[0:256], emb_norm[256:512], emb_norm[512:768],
        emb_norm[512:768], emb_norm[768:1024], emb_norm[1024:2048],
        emb_norm[2048:3072],
    )
    e_bd = jnp.zeros((N_PROJ, N_OUT), dtype=jnp.float32)
    for g, (off, width, _) in enumerate(GROUPS):
        e_bd = e_bd.at[64 * g:64 * (g + 1), off:off + width].set(
            emb_chunks[g].T)
    e_bd = e_bd.astype(jnp.bfloat16)

    grid = (n_tok // TILE,)
    out = pl.pallas_call(
        _router_kernel,
        grid=grid,
        in_specs=[
            pl.BlockSpec((TILE, d), lambda i: (i, 0)),
            pl.BlockSpec((d, N_PROJ), lambda i: (0, 0)),
            pl.BlockSpec((1, N_PROJ), lambda i: (0, 0)),
            pl.BlockSpec((N_PROJ, N_OUT), lambda i: (0, 0)),
        ],
        out_specs=pl.BlockSpec((TILE, N_OUT), lambda i: (i, 0)),
        out_shape=jax.ShapeDtypeStruct((n_tok, N_OUT), jnp.float32),
    )(xf, w_cat, b_cat, e_bd)
    return out.reshape(b, s, N_OUT)
